# baseline (speedup 1.0000x reference)
import functools

import jax
import jax.numpy as jnp
from jax import lax
from jax.experimental import pallas as pl
from jax.experimental.pallas import tpu as pltpu
import jax.experimental.pallas.tpu_sc as plsc

N = 10000
E = 320000
D = 128
C = 10

NT = 10112
DW = 144
NC = 2
NS = 16
NW = NC * NS
BLK = 128
J = 80
PH = 10
PB = J // PH
EPAD = NW * J * BLK

ROWS_PER_TILE = NT // NS
CP = 79
ZROW = NT - CP


def _sc_aggregate(aug, src_blk, dst_blk):
    mesh = plsc.VectorSubcoreMesh(core_axis_name="c", subcore_axis_name="s")

    @functools.partial(
        pl.kernel,
        out_type=jax.ShapeDtypeStruct((NC, NT, DW), jnp.float32),
        mesh=mesh,
        scratch_types=[
            pltpu.MemorySpace.VMEM_SHARED((NT, DW), jnp.float32),
            pltpu.MemorySpace.VMEM((PB, BLK), jnp.int32),
            pltpu.MemorySpace.VMEM((PB, BLK), jnp.int32),
            pltpu.MemorySpace.VMEM((BLK, DW), jnp.float32),
            pltpu.MemorySpace.VMEM((BLK, DW), jnp.float32),
            pltpu.SemaphoreType.DMA,
            pltpu.SemaphoreType.DMA,
        ],
        compiler_params=pltpu.CompilerParams(use_tc_tiling_on_sc=False),
    )
    def body(aug_hbm, src_hbm, dst_hbm, out_hbm, acc_sh, src_v, dst_v,
             rows_a, rows_b, sem_a, sem_b):
        cid = lax.axis_index("c")
        sid = lax.axis_index("s")
        wid = cid * NS + sid

        pltpu.sync_copy(aug_hbm.at[pl.ds(ZROW, CP)], rows_a.at[pl.ds(0, CP)])
        row0 = sid * ROWS_PER_TILE
        for i in range(ROWS_PER_TILE // CP):
            pltpu.sync_copy(rows_a.at[pl.ds(0, CP)],
                            acc_sh.at[pl.ds(row0 + i * CP, CP)])

        plsc.subcore_barrier()

        def phase(p, carry):
            pltpu.sync_copy(src_hbm.at[wid, pl.ds(p * PB, PB)], src_v)
            pltpu.sync_copy(dst_hbm.at[wid, pl.ds(p * PB, PB)], dst_v)
            pltpu.async_copy(aug_hbm.at[src_v.at[0]], rows_a, sem_a)

            def step(t, c2):
                j = t * 2
                pltpu.make_async_copy(aug_hbm.at[src_v.at[j]], rows_a, sem_a).wait()
                pltpu.async_copy(aug_hbm.at[src_v.at[j + 1]], rows_b, sem_b)
                pltpu.sync_copy(rows_a, acc_sh.at[dst_v.at[j]], add=True)
                pltpu.make_async_copy(aug_hbm.at[src_v.at[j + 1]], rows_b, sem_b).wait()

                @pl.when(t + 1 < PB // 2)
                def _next():
                    pltpu.async_copy(aug_hbm.at[src_v.at[j + 2]], rows_a, sem_a)

                pltpu.sync_copy(rows_b, acc_sh.at[dst_v.at[j + 1]], add=True)
                return c2

            lax.fori_loop(0, PB // 2, step, 0)
            return carry

        lax.fori_loop(0, PH, phase, 0)

        plsc.subcore_barrier()

        for i in range(ROWS_PER_TILE // CP):
            r = row0 + i * CP
            pltpu.sync_copy(acc_sh.at[pl.ds(r, CP)], rows_a.at[pl.ds(0, CP)])
            pltpu.sync_copy(rows_a.at[pl.ds(0, CP)],
                            out_hbm.at[cid, pl.ds(r, CP)])

    return body(aug, src_blk, dst_blk)


ROWB = 1264


def _tc_body(aug_ref, p_ref, w_ref, wc_ref, bc_ref, out_ref, acc_ref):
    i = pl.program_id(0)

    @pl.when(i == 0)
    def _init():
        acc_ref[...] = jnp.zeros_like(acc_ref)

    blk = aug_ref[...]
    x = blk[:, :D]
    p = p_ref[0] + p_ref[1]
    deg = jnp.clip(p[:, D:D + 1], 1.0, None)
    neigh = p[:, :D] / deg
    w = w_ref[...]
    h = x @ w[:D] + neigh @ w[D:]
    h = jnp.maximum(h, 0.0)
    acc_ref[...] += jnp.sum(h, axis=0, keepdims=True)

    @pl.when(i == pl.num_programs(0) - 1)
    def _fin():
        ge = acc_ref[...]
        scores = lax.dot_general(ge, wc_ref[...], (((1,), (1,)), ((), ())))
        out_ref[...] = scores + bc_ref[...]


def _tc_readout(aug, partials, W, Wc, bc2):
    grid = (NT // ROWB,)
    return pl.pallas_call(
        _tc_body,
        grid=grid,
        in_specs=[
            pl.BlockSpec((ROWB, DW), lambda i: (i, 0)),
            pl.BlockSpec((NC, ROWB, DW), lambda i: (0, i, 0)),
            pl.BlockSpec((2 * D, D), lambda i: (0, 0)),
            pl.BlockSpec((C, D), lambda i: (0, 0)),
            pl.BlockSpec((1, C), lambda i: (0, 0)),
        ],
        out_specs=pl.BlockSpec((1, C), lambda i: (0, 0)),
        out_shape=jax.ShapeDtypeStruct((1, C), jnp.float32),
        scratch_shapes=[pltpu.VMEM((1, D), jnp.float32)],
    )(aug, partials, W, Wc, bc2)


def kernel(features, edge_index, W, Wc, bc):
    f32 = jnp.float32
    top = jnp.concatenate(
        [features,
         jnp.ones((N, 1), f32),
         jnp.zeros((N, DW - D - 1), f32)], axis=1)
    aug = jnp.concatenate([top, jnp.zeros((NT - N, DW), f32)], axis=0)

    pad = EPAD - E
    src = jnp.concatenate([edge_index[0], jnp.full((pad,), N, jnp.int32)])
    dst = jnp.concatenate([edge_index[1], jnp.full((pad,), N, jnp.int32)])
    src_blk = src.reshape(NW, J, BLK)
    dst_blk = dst.reshape(NW, J, BLK)

    partials = _sc_aggregate(aug, src_blk, dst_blk)
    scores = _tc_readout(aug, partials, W, Wc, bc.reshape(1, C))
    return scores


# speedup vs baseline: 1.4381x; 1.4381x over previous
import functools

import jax
import jax.numpy as jnp
from jax import lax
from jax.experimental import pallas as pl
from jax.experimental.pallas import tpu as pltpu
import jax.experimental.pallas.tpu_sc as plsc

N = 10000
E = 320000
D = 128
C = 10

NT = 10240
DW = 144
NC = 2
NS = 16
NW = NC * NS
BLK = 64
J = 158
EPAD = NW * J * BLK

ROWS_PER_TILE = NT // NS
CP = 64
ZROW = NT - CP


def _sc_aggregate(aug, src_blk, dst_blk):
    mesh = plsc.VectorSubcoreMesh(core_axis_name="c", subcore_axis_name="s")

    @functools.partial(
        pl.kernel,
        out_type=jax.ShapeDtypeStruct((NC, NT, DW), jnp.float32),
        mesh=mesh,
        scratch_types=[
            pltpu.MemorySpace.VMEM_SHARED((NT, DW), jnp.float32),
            pltpu.MemorySpace.VMEM((J, BLK), jnp.int32),
            pltpu.MemorySpace.VMEM((J, BLK), jnp.int32),
            pltpu.MemorySpace.VMEM((BLK, DW), jnp.float32),
            pltpu.MemorySpace.VMEM((BLK, DW), jnp.float32),
            pltpu.SemaphoreType.DMA,
            pltpu.SemaphoreType.DMA,
        ],
        compiler_params=pltpu.CompilerParams(use_tc_tiling_on_sc=False),
    )
    def body(aug_hbm, src_hbm, dst_hbm, out_hbm, acc_sh, src_v, dst_v,
             rows_a, rows_b, sem_a, sem_b):
        cid = lax.axis_index("c")
        sid = lax.axis_index("s")
        wid = cid * NS + sid

        pltpu.sync_copy(aug_hbm.at[pl.ds(ZROW, CP)], rows_a.at[pl.ds(0, CP)])
        row0 = sid * ROWS_PER_TILE
        for i in range(ROWS_PER_TILE // CP):
            pltpu.sync_copy(rows_a.at[pl.ds(0, CP)],
                            acc_sh.at[pl.ds(row0 + i * CP, CP)])

        pltpu.sync_copy(src_hbm.at[wid], src_v)
        pltpu.sync_copy(dst_hbm.at[wid], dst_v)

        plsc.subcore_barrier()

        pltpu.async_copy(aug_hbm.at[src_v.at[0]], rows_a, sem_a)

        def step(t, c2):
            j = t * 2
            pltpu.make_async_copy(aug_hbm.at[src_v.at[j]], rows_a, sem_a).wait()
            pltpu.async_copy(aug_hbm.at[src_v.at[j + 1]], rows_b, sem_b)
            pltpu.sync_copy(rows_a, acc_sh.at[dst_v.at[j]], add=True)
            pltpu.make_async_copy(aug_hbm.at[src_v.at[j + 1]], rows_b, sem_b).wait()

            @pl.when(t + 1 < J // 2)
            def _next():
                pltpu.async_copy(aug_hbm.at[src_v.at[j + 2]], rows_a, sem_a)

            pltpu.sync_copy(rows_b, acc_sh.at[dst_v.at[j + 1]], add=True)
            return c2

        lax.fori_loop(0, J // 2, step, 0)

        plsc.subcore_barrier()

        for i in range(ROWS_PER_TILE // CP):
            r = row0 + i * CP
            pltpu.sync_copy(acc_sh.at[pl.ds(r, CP)], rows_a.at[pl.ds(0, CP)])
            pltpu.sync_copy(rows_a.at[pl.ds(0, CP)],
                            out_hbm.at[cid, pl.ds(r, CP)])

    return body(aug, src_blk, dst_blk)


ROWB = 1024


def _tc_body(aug_ref, p_ref, w_ref, wc_ref, bc_ref, out_ref, acc_ref):
    i = pl.program_id(0)

    @pl.when(i == 0)
    def _init():
        acc_ref[...] = jnp.zeros_like(acc_ref)

    blk = aug_ref[...]
    x = blk[:, :D]
    p = p_ref[0] + p_ref[1]
    deg = jnp.clip(p[:, D:D + 1], 1.0, None)
    neigh = p[:, :D] / deg
    w = w_ref[...]
    h = x @ w[:D] + neigh @ w[D:]
    h = jnp.maximum(h, 0.0)
    acc_ref[...] += jnp.sum(h, axis=0, keepdims=True)

    @pl.when(i == pl.num_programs(0) - 1)
    def _fin():
        ge = acc_ref[...]
        scores = lax.dot_general(ge, wc_ref[...], (((1,), (1,)), ((), ())))
        out_ref[...] = scores + bc_ref[...]


def _tc_readout(aug, partials, W, Wc, bc2):
    grid = (NT // ROWB,)
    return pl.pallas_call(
        _tc_body,
        grid=grid,
        in_specs=[
            pl.BlockSpec((ROWB, DW), lambda i: (i, 0)),
            pl.BlockSpec((NC, ROWB, DW), lambda i: (0, i, 0)),
            pl.BlockSpec((2 * D, D), lambda i: (0, 0)),
            pl.BlockSpec((C, D), lambda i: (0, 0)),
            pl.BlockSpec((1, C), lambda i: (0, 0)),
        ],
        out_specs=pl.BlockSpec((1, C), lambda i: (0, 0)),
        out_shape=jax.ShapeDtypeStruct((1, C), jnp.float32),
        scratch_shapes=[pltpu.VMEM((1, D), jnp.float32)],
    )(aug, partials, W, Wc, bc2)


def kernel(features, edge_index, W, Wc, bc):
    f32 = jnp.float32
    top = jnp.concatenate(
        [features,
         jnp.ones((N, 1), f32),
         jnp.zeros((N, DW - D - 1), f32)], axis=1)
    aug = jnp.concatenate([top, jnp.zeros((NT - N, DW), f32)], axis=0)

    pad = EPAD - E
    src = jnp.concatenate([edge_index[0], jnp.full((pad,), N, jnp.int32)])
    dst = jnp.concatenate([edge_index[1], jnp.full((pad,), N, jnp.int32)])
    src_blk = src.reshape(NW, J, BLK)
    dst_blk = dst.reshape(NW, J, BLK)

    partials = _sc_aggregate(aug, src_blk, dst_blk)
    scores = _tc_readout(aug, partials, W, Wc, bc.reshape(1, C))
    return scores
